# Initial kernel scaffold; baseline (speedup 1.0000x reference)
import jax
import jax.numpy as jnp
from jax.experimental import pallas as pl


def kernel(x, edge_index, W1, b1, Wmu, bmu, Wls, bls):
    raise NotImplementedError("write your pallas kernel here")


# speedup vs baseline: 32.4870x; 32.4870x over previous
import functools

import jax
import jax.numpy as jnp
from jax import lax
from jax.experimental import pallas as pl
from jax.experimental.pallas import tpu as pltpu
from jax.experimental.pallas import tpu_sc as plsc

NC = 2
NS = 16
NW = NC * NS
WIN = 128
RB = 1024


def _mesh():
    return plsc.VectorSubcoreMesh(
        core_axis_name="c", subcore_axis_name="s",
        num_cores=NC, num_subcores=NS)


@functools.lru_cache(maxsize=None)
def _make_sc_degree(n_pad, n_win):
    chunk = n_pad // NS

    @functools.partial(
        pl.kernel,
        out_type=jax.ShapeDtypeStruct((NC, n_pad), jnp.float32),
        mesh=_mesh(),
        scratch_types=[
            pltpu.VMEM((n_win, WIN), jnp.int32),
            pltpu.VMEM((WIN,), jnp.float32),
            pltpu.VMEM((chunk,), jnp.float32),
            pltpu.VMEM_SHARED((n_pad,), jnp.float32),
        ],
    )
    def deg_kernel(dst_hbm, out_hbm, idx_v, ones_v, zeros_v, acc_sh):
        c = lax.axis_index("c")
        s = lax.axis_index("s")
        wid = s * NC + c
        pltpu.sync_copy(dst_hbm.at[wid], idx_v)
        for k in range(WIN // 16):
            ones_v[pl.ds(k * 16, 16)] = jnp.full((16,), 1.0, jnp.float32)

        def zfill(i, carry):
            zeros_v[pl.ds(i * 16, 16)] = jnp.zeros((16,), jnp.float32)
            return carry

        lax.fori_loop(0, chunk // 16, zfill, 0)
        pltpu.sync_copy(zeros_v, acc_sh.at[pl.ds(s * chunk, chunk)])
        plsc.subcore_barrier()

        def body(j, carry):
            pltpu.sync_copy(ones_v, acc_sh.at[idx_v.at[j]], add=True)
            return carry

        lax.fori_loop(0, n_win, body, 0)
        plsc.subcore_barrier()
        pltpu.sync_copy(acc_sh.at[pl.ds(s * chunk, chunk)],
                        out_hbm.at[c, pl.ds(s * chunk, chunk)])

    return deg_kernel


@functools.lru_cache(maxsize=None)
def _make_sc_propagate(n_pad, n_win, d):
    chunk = n_pad // NS

    @functools.partial(
        pl.kernel,
        out_type=jax.ShapeDtypeStruct((NC, n_pad, d), jnp.float32),
        mesh=_mesh(),
        compiler_params=pltpu.CompilerParams(use_tc_tiling_on_sc=False),
        scratch_types=[
            pltpu.VMEM((n_win, WIN), jnp.int32),
            pltpu.VMEM((n_win, WIN), jnp.int32),
            pltpu.VMEM((WIN, d), jnp.float32),
            pltpu.VMEM((16, d), jnp.float32),
            pltpu.VMEM_SHARED((n_pad, d), jnp.float32),
            pltpu.SemaphoreType.DMA,
        ],
    )
    def prop_kernel(z_hbm, src_hbm, dst_hbm, out_hbm,
                    src_v, dst_v, rows_v, zb_v, acc_sh, sem):
        c = lax.axis_index("c")
        s = lax.axis_index("s")
        wid = s * NC + c
        pltpu.sync_copy(src_hbm.at[wid], src_v)
        pltpu.sync_copy(dst_hbm.at[wid], dst_v)
        for i in range(16):
            for k in range(d // 16):
                zb_v[i, pl.ds(k * 16, 16)] = jnp.zeros((16,), jnp.float32)

        def zfill(i, carry):
            pltpu.sync_copy(zb_v, acc_sh.at[pl.ds(s * chunk + i * 16, 16)])
            return carry

        lax.fori_loop(0, chunk // 16, zfill, 0)
        plsc.subcore_barrier()

        def body(j, carry):
            pltpu.async_copy(z_hbm.at[src_v.at[j]], rows_v, sem).wait()
            pltpu.sync_copy(rows_v, acc_sh.at[dst_v.at[j]], add=True)
            return carry

        lax.fori_loop(0, n_win, body, 0)
        plsc.subcore_barrier()
        pltpu.sync_copy(acc_sh.at[pl.ds(s * chunk, chunk)],
                        out_hbm.at[c, pl.ds(s * chunk, chunk)])

    return prop_kernel


def _dinv_block(deg_blk, pid, n):
    dtot = jnp.sum(deg_blk, axis=1, keepdims=True) + 1.0
    rows = pid * RB + lax.broadcasted_iota(jnp.int32, (RB, 1), 0)
    return jnp.where(rows < n, lax.rsqrt(dtot), 0.0)


@functools.lru_cache(maxsize=None)
def _make_tc1(n, n_pad, in_ch, h):
    def body(x_ref, w_ref, deg_ref, zp_ref):
        dinv = _dinv_block(deg_ref[...], pl.program_id(0), n)
        xw = jnp.dot(x_ref[...], w_ref[...],
                     preferred_element_type=jnp.float32)
        zp_ref[...] = dinv * xw

    return pl.pallas_call(
        body,
        grid=(n_pad // RB,),
        in_specs=[
            pl.BlockSpec((RB, in_ch), lambda i: (i, 0)),
            pl.BlockSpec((in_ch, h), lambda i: (0, 0)),
            pl.BlockSpec((RB, NC), lambda i: (i, 0)),
        ],
        out_specs=pl.BlockSpec((RB, h), lambda i: (i, 0)),
        out_shape=jax.ShapeDtypeStruct((n_pad, h), jnp.float32),
    )


@functools.lru_cache(maxsize=None)
def _make_tc2(n, n_pad, h, o2):
    def body(p_ref, zp_ref, deg_ref, b1_ref, w_ref, tp_ref):
        dinv = _dinv_block(deg_ref[...], pl.program_id(0), n)
        hcur = dinv * (p_ref[0] + p_ref[1] + zp_ref[...]) + b1_ref[...]
        t = jnp.dot(hcur, w_ref[...], preferred_element_type=jnp.float32)
        tp_ref[...] = dinv * t

    return pl.pallas_call(
        body,
        grid=(n_pad // RB,),
        in_specs=[
            pl.BlockSpec((NC, RB, h), lambda i: (0, i, 0)),
            pl.BlockSpec((RB, h), lambda i: (i, 0)),
            pl.BlockSpec((RB, NC), lambda i: (i, 0)),
            pl.BlockSpec((1, h), lambda i: (0, 0)),
            pl.BlockSpec((h, o2), lambda i: (0, 0)),
        ],
        out_specs=pl.BlockSpec((RB, o2), lambda i: (i, 0)),
        out_shape=jax.ShapeDtypeStruct((n_pad, o2), jnp.float32),
    )


@functools.lru_cache(maxsize=None)
def _make_tc3(n, n_pad, o2):
    def body(q_ref, tp_ref, deg_ref, b_ref, out_ref):
        dinv = _dinv_block(deg_ref[...], pl.program_id(0), n)
        u = dinv * (q_ref[0] + q_ref[1] + tp_ref[...])
        out_ref[...] = u + b_ref[...]

    return pl.pallas_call(
        body,
        grid=(n_pad // RB,),
        in_specs=[
            pl.BlockSpec((NC, RB, o2), lambda i: (0, i, 0)),
            pl.BlockSpec((RB, o2), lambda i: (i, 0)),
            pl.BlockSpec((RB, NC), lambda i: (i, 0)),
            pl.BlockSpec((1, o2), lambda i: (0, 0)),
        ],
        out_specs=pl.BlockSpec((RB, o2), lambda i: (i, 0)),
        out_shape=jax.ShapeDtypeStruct((n_pad, o2), jnp.float32),
    )


def kernel(x, edge_index, W1, b1, Wmu, bmu, Wls, bls):
    n, in_ch = x.shape
    h = W1.shape[1]
    o = Wmu.shape[1]
    o2 = 2 * o
    e = edge_index.shape[1]

    n_pad = -(-(n + 16) // (16 * NS)) * (16 * NS)
    n_win = -(-e // (NW * WIN))
    e_pad = n_win * NW * WIN

    src = edge_index[0]
    dst = edge_index[1]
    padidx = n + (jnp.arange(e_pad - e, dtype=jnp.int32) % 16)
    src_p = jnp.concatenate([src, padidx]).reshape(NW, n_win, WIN)
    dst_p = jnp.concatenate([dst, padidx]).reshape(NW, n_win, WIN)
    x_p = jnp.pad(x, ((0, n_pad - n), (0, 0)))

    deg = _make_sc_degree(n_pad, n_win)(dst_p)
    deg_t = deg.T

    zp = _make_tc1(n, n_pad, in_ch, h)(x_p, W1, deg_t)

    prop = _make_sc_propagate(n_pad, n_win, h)
    p = prop(zp, src_p, dst_p)

    w_cat = jnp.concatenate([Wmu, Wls], axis=1)
    tp = _make_tc2(n, n_pad, h, o2)(p, zp, deg_t, b1.reshape(1, h), w_cat)

    q = _make_sc_propagate(n_pad, n_win, o2)(tp, src_p, dst_p)

    b_cat = jnp.concatenate([bmu, bls]).reshape(1, o2)
    res = _make_tc3(n, n_pad, o2)(q, tp, deg_t, b_cat)

    return (res[:n, :o], res[:n, o:])


# speedup vs baseline: 53.9691x; 1.6612x over previous
import functools

import jax
import jax.numpy as jnp
from jax import lax
from jax.experimental import pallas as pl
from jax.experimental.pallas import tpu as pltpu
from jax.experimental.pallas import tpu_sc as plsc

NC = 2
NS = 16
NW = NC * NS
WIN = 128
RB = 1024


def _mesh():
    return plsc.VectorSubcoreMesh(
        core_axis_name="c", subcore_axis_name="s",
        num_cores=NC, num_subcores=NS)


@functools.lru_cache(maxsize=None)
def _make_sc_degree(n_pad, n_win):
    chunk = n_pad // NS

    @functools.partial(
        pl.kernel,
        out_type=jax.ShapeDtypeStruct((NC, n_pad), jnp.float32),
        mesh=_mesh(),
        scratch_types=[
            pltpu.VMEM((n_win, WIN), jnp.int32),
            pltpu.VMEM((WIN,), jnp.float32),
            pltpu.VMEM((chunk,), jnp.float32),
            pltpu.VMEM_SHARED((n_pad,), jnp.float32),
        ],
    )
    def deg_kernel(edges_hbm, out_hbm, idx_v, ones_v, zeros_v, acc_sh):
        c = lax.axis_index("c")
        s = lax.axis_index("s")
        wid = s * NC + c
        pltpu.sync_copy(edges_hbm.at[1, wid], idx_v)
        for k in range(WIN // 16):
            ones_v[pl.ds(k * 16, 16)] = jnp.full((16,), 1.0, jnp.float32)

        def zfill(i, carry):
            zeros_v[pl.ds(i * 16, 16)] = jnp.zeros((16,), jnp.float32)
            return carry

        lax.fori_loop(0, chunk // 16, zfill, 0)
        pltpu.sync_copy(zeros_v, acc_sh.at[pl.ds(s * chunk, chunk)])
        plsc.subcore_barrier()

        def body(j, carry):
            pltpu.sync_copy(ones_v, acc_sh.at[idx_v.at[j]], add=True)
            return carry

        lax.fori_loop(0, n_win, body, 0)
        plsc.subcore_barrier()
        pltpu.sync_copy(acc_sh.at[pl.ds(s * chunk, chunk)],
                        out_hbm.at[c, pl.ds(s * chunk, chunk)])

    return deg_kernel


NBUF = 4


@functools.lru_cache(maxsize=None)
def _make_sc_propagate(n_pad, n_win, d):
    chunk = n_pad // NS
    zrows = 64
    assert n_win % NBUF == 0 and chunk % zrows == 0

    @functools.partial(
        pl.kernel,
        out_type=jax.ShapeDtypeStruct((NC, n_pad, d), jnp.float32),
        mesh=_mesh(),
        compiler_params=pltpu.CompilerParams(use_tc_tiling_on_sc=False),
        scratch_types=[
            pltpu.VMEM((2, n_win, WIN), jnp.int32),
            pltpu.VMEM((NBUF, WIN, d), jnp.float32),
            pltpu.VMEM((zrows, d), jnp.float32),
            pltpu.VMEM_SHARED((n_pad, d), jnp.float32),
            pltpu.SemaphoreType.DMA,
        ],
    )
    def prop_kernel(z_hbm, edges_hbm, out_hbm,
                    idx_v, rows_v, zb_v, acc_sh, gsem):
        c = lax.axis_index("c")
        s = lax.axis_index("s")
        wid = s * NC + c
        pltpu.sync_copy(edges_hbm.at[0, wid], idx_v.at[0])
        pltpu.sync_copy(edges_hbm.at[1, wid], idx_v.at[1])
        for k in range(zrows * d // 16):
            zb_v[k // (d // 16), pl.ds((k % (d // 16)) * 16, 16)] = (
                jnp.zeros((16,), jnp.float32))

        def zfill(i, carry):
            pltpu.sync_copy(
                zb_v, acc_sh.at[pl.ds(s * chunk + i * zrows, zrows)])
            return carry

        lax.fori_loop(0, chunk // zrows, zfill, 0)
        plsc.subcore_barrier()

        src_v = idx_v.at[0]
        dst_v = idx_v.at[1]

        def gather(j, b):
            pltpu.async_copy(z_hbm.at[src_v.at[j]], rows_v.at[b], gsem)

        def gather_wait(j, b):
            pltpu.make_async_copy(
                z_hbm.at[src_v.at[j]], rows_v.at[b], gsem).wait()

        def step(j, b, prefetch):
            gather_wait(j, b)
            pltpu.sync_copy(rows_v.at[b], acc_sh.at[dst_v.at[j]], add=True)
            if prefetch:
                gather(j + NBUF, b)

        for b in range(NBUF):
            gather(b, b)

        def group(g, carry):
            for b in range(NBUF):
                step(g * NBUF + b, b, True)
            return carry

        n_grp = n_win // NBUF
        lax.fori_loop(0, n_grp - 1, group, 0)
        for b in range(NBUF):
            step((n_grp - 1) * NBUF + b, b, False)
        plsc.subcore_barrier()
        pltpu.sync_copy(acc_sh.at[pl.ds(s * chunk, chunk)],
                        out_hbm.at[c, pl.ds(s * chunk, chunk)])

    return prop_kernel


def _dinv_block(deg_blk, pid, n):
    dtot = jnp.sum(deg_blk, axis=1, keepdims=True) + 1.0
    rows = pid * RB + lax.broadcasted_iota(jnp.int32, (RB, 1), 0)
    return jnp.where(rows < n, lax.rsqrt(dtot), 0.0)


@functools.lru_cache(maxsize=None)
def _make_tc1(n, n_pad, in_ch, h):
    def body(x_ref, w_ref, deg_ref, zp_ref):
        dinv = _dinv_block(deg_ref[...], pl.program_id(0), n)
        xw = jnp.dot(x_ref[...], w_ref[...],
                     preferred_element_type=jnp.float32)
        zp_ref[...] = dinv * xw

    return pl.pallas_call(
        body,
        grid=(n_pad // RB,),
        in_specs=[
            pl.BlockSpec((RB, in_ch), lambda i: (i, 0)),
            pl.BlockSpec((in_ch, h), lambda i: (0, 0)),
            pl.BlockSpec((RB, NC), lambda i: (i, 0)),
        ],
        out_specs=pl.BlockSpec((RB, h), lambda i: (i, 0)),
        out_shape=jax.ShapeDtypeStruct((n_pad, h), jnp.float32),
    )


@functools.lru_cache(maxsize=None)
def _make_tc2(n, n_pad, h, o2):
    def body(p_ref, zp_ref, deg_ref, b1_ref, w_ref, tp_ref):
        dinv = _dinv_block(deg_ref[...], pl.program_id(0), n)
        hcur = dinv * (p_ref[0] + p_ref[1] + zp_ref[...]) + b1_ref[...]
        t = jnp.dot(hcur, w_ref[...], preferred_element_type=jnp.float32)
        tp_ref[...] = dinv * t

    return pl.pallas_call(
        body,
        grid=(n_pad // RB,),
        in_specs=[
            pl.BlockSpec((NC, RB, h), lambda i: (0, i, 0)),
            pl.BlockSpec((RB, h), lambda i: (i, 0)),
            pl.BlockSpec((RB, NC), lambda i: (i, 0)),
            pl.BlockSpec((1, h), lambda i: (0, 0)),
            pl.BlockSpec((h, o2), lambda i: (0, 0)),
        ],
        out_specs=pl.BlockSpec((RB, o2), lambda i: (i, 0)),
        out_shape=jax.ShapeDtypeStruct((n_pad, o2), jnp.float32),
    )


@functools.lru_cache(maxsize=None)
def _make_tc3(n, n_pad, o2):
    def body(q_ref, tp_ref, deg_ref, b_ref, out_ref):
        dinv = _dinv_block(deg_ref[...], pl.program_id(0), n)
        u = dinv * (q_ref[0] + q_ref[1] + tp_ref[...])
        out_ref[...] = u + b_ref[...]

    return pl.pallas_call(
        body,
        grid=(n_pad // RB,),
        in_specs=[
            pl.BlockSpec((NC, RB, o2), lambda i: (0, i, 0)),
            pl.BlockSpec((RB, o2), lambda i: (i, 0)),
            pl.BlockSpec((RB, NC), lambda i: (i, 0)),
            pl.BlockSpec((1, o2), lambda i: (0, 0)),
        ],
        out_specs=pl.BlockSpec((RB, o2), lambda i: (i, 0)),
        out_shape=jax.ShapeDtypeStruct((n_pad, o2), jnp.float32),
    )


def kernel(x, edge_index, W1, b1, Wmu, bmu, Wls, bls):
    n, in_ch = x.shape
    h = W1.shape[1]
    o = Wmu.shape[1]
    o2 = 2 * o
    e = edge_index.shape[1]

    n_pad = -(-(n + 16) // (16 * NS)) * (16 * NS)
    n_win = -(-(-(-e // (NW * WIN))) // NBUF) * NBUF
    e_pad = n_win * NW * WIN

    padidx = n + (jnp.arange(e_pad - e, dtype=jnp.int32) % (n_pad - n))
    edges = jnp.concatenate(
        [edge_index, jnp.stack([padidx, padidx])], axis=1
    ).reshape(2, NW, n_win, WIN)
    x_p = jnp.pad(x, ((0, n_pad - n), (0, 0)))

    deg = _make_sc_degree(n_pad, n_win)(edges)
    deg_t = deg.T

    zp = _make_tc1(n, n_pad, in_ch, h)(x_p, W1, deg_t)

    prop = _make_sc_propagate(n_pad, n_win, h)
    p = prop(zp, edges)

    w_cat = jnp.concatenate([Wmu, Wls], axis=1)
    tp = _make_tc2(n, n_pad, h, o2)(p, zp, deg_t, b1.reshape(1, h), w_cat)

    q = _make_sc_propagate(n_pad, n_win, o2)(tp, edges)

    b_cat = jnp.concatenate([bmu, bls]).reshape(1, o2)
    res = _make_tc3(n, n_pad, o2)(q, tp, deg_t, b_cat)

    return (res[:n, :o], res[:n, o:])


# speedup vs baseline: 56.7675x; 1.0519x over previous
import functools

import jax
import jax.numpy as jnp
from jax import lax
from jax.experimental import pallas as pl
from jax.experimental.pallas import tpu as pltpu
from jax.experimental.pallas import tpu_sc as plsc

NC = 2
NS = 16
NW = NC * NS
WIN = 128
RB = 1024


def _mesh():
    return plsc.VectorSubcoreMesh(
        core_axis_name="c", subcore_axis_name="s",
        num_cores=NC, num_subcores=NS)


@functools.lru_cache(maxsize=None)
def _make_sc_degree(n_pad, n_win):
    chunk = n_pad // NS

    @functools.partial(
        pl.kernel,
        out_type=jax.ShapeDtypeStruct((NC, n_pad), jnp.float32),
        mesh=_mesh(),
        scratch_types=[
            pltpu.VMEM((n_win, WIN), jnp.int32),
            pltpu.VMEM((WIN,), jnp.float32),
            pltpu.VMEM((chunk,), jnp.float32),
            pltpu.VMEM_SHARED((n_pad,), jnp.float32),
        ],
    )
    def deg_kernel(edges_hbm, out_hbm, idx_v, ones_v, zeros_v, acc_sh):
        c = lax.axis_index("c")
        s = lax.axis_index("s")
        wid = s * NC + c
        pltpu.sync_copy(edges_hbm.at[1, wid], idx_v)
        for k in range(WIN // 16):
            ones_v[pl.ds(k * 16, 16)] = jnp.full((16,), 1.0, jnp.float32)

        def zfill(i, carry):
            zeros_v[pl.ds(i * 16, 16)] = jnp.zeros((16,), jnp.float32)
            return carry

        lax.fori_loop(0, chunk // 16, zfill, 0)
        pltpu.sync_copy(zeros_v, acc_sh.at[pl.ds(s * chunk, chunk)])
        plsc.subcore_barrier()

        def body(j, carry):
            pltpu.sync_copy(ones_v, acc_sh.at[idx_v.at[j]], add=True)
            return carry

        lax.fori_loop(0, n_win, body, 0)
        plsc.subcore_barrier()
        pltpu.sync_copy(acc_sh.at[pl.ds(s * chunk, chunk)],
                        out_hbm.at[c, pl.ds(s * chunk, chunk)])

    return deg_kernel


NBUF = 4


@functools.lru_cache(maxsize=None)
def _make_sc_propagate(n_pad, n_win, d):
    chunk = n_pad // NS
    zrows = 64
    assert n_win % NBUF == 0 and chunk % zrows == 0

    @functools.partial(
        pl.kernel,
        out_type=jax.ShapeDtypeStruct((NC, n_pad, d), jnp.float32),
        mesh=_mesh(),
        compiler_params=pltpu.CompilerParams(use_tc_tiling_on_sc=False),
        scratch_types=[
            pltpu.VMEM((2, n_win, WIN), jnp.int32),
            pltpu.VMEM((NBUF, WIN, d), jnp.float32),
            pltpu.VMEM((zrows, d), jnp.float32),
            pltpu.VMEM_SHARED((n_pad, d), jnp.float32),
            pltpu.SemaphoreType.DMA,
        ],
    )
    def prop_kernel(z_hbm, edges_hbm, out_hbm,
                    idx_v, rows_v, zb_v, acc_sh, gsem):
        c = lax.axis_index("c")
        s = lax.axis_index("s")
        wid = s * NC + c
        pltpu.sync_copy(edges_hbm.at[0, wid], idx_v.at[0])
        pltpu.sync_copy(edges_hbm.at[1, wid], idx_v.at[1])
        for k in range(zrows * d // 16):
            zb_v[k // (d // 16), pl.ds((k % (d // 16)) * 16, 16)] = (
                jnp.zeros((16,), jnp.float32))

        def zfill(i, carry):
            pltpu.sync_copy(
                zb_v, acc_sh.at[pl.ds(s * chunk + i * zrows, zrows)])
            return carry

        lax.fori_loop(0, chunk // zrows, zfill, 0)
        plsc.subcore_barrier()

        src_v = idx_v.at[0]
        dst_v = idx_v.at[1]

        def gather(j, b):
            pltpu.async_copy(z_hbm.at[src_v.at[j]], rows_v.at[b], gsem)

        def gather_wait(j, b):
            pltpu.make_async_copy(
                z_hbm.at[src_v.at[j]], rows_v.at[b], gsem).wait()

        def step(j, b, prefetch):
            gather_wait(j, b)
            pltpu.sync_copy(rows_v.at[b], acc_sh.at[dst_v.at[j]], add=True)
            if prefetch:
                gather(j + NBUF, b)

        for b in range(NBUF):
            gather(b, b)

        def group(g, carry):
            for b in range(NBUF):
                step(g * NBUF + b, b, True)
            return carry

        n_grp = n_win // NBUF
        lax.fori_loop(0, n_grp - 1, group, 0)
        for b in range(NBUF):
            step((n_grp - 1) * NBUF + b, b, False)
        plsc.subcore_barrier()
        pltpu.sync_copy(acc_sh.at[pl.ds(s * chunk, chunk)],
                        out_hbm.at[c, pl.ds(s * chunk, chunk)])

    return prop_kernel


def _dinv_block(deg_blk, pid, n):
    dtot = (jnp.sum(deg_blk, axis=0) + 1.0).reshape(RB, 1)
    rows = pid * RB + lax.broadcasted_iota(jnp.int32, (RB, 1), 0)
    return jnp.where(rows < n, lax.rsqrt(dtot), 0.0)


@functools.lru_cache(maxsize=None)
def _make_tc1(n, n_pad, in_ch, h):
    def body(x_ref, w_ref, deg_ref, zp_ref):
        dinv = _dinv_block(deg_ref[...], pl.program_id(0), n)
        xw = jnp.dot(x_ref[...], w_ref[...],
                     preferred_element_type=jnp.float32)
        zp_ref[...] = dinv * xw

    return pl.pallas_call(
        body,
        grid=(n_pad // RB,),
        in_specs=[
            pl.BlockSpec((RB, in_ch), lambda i: (i, 0)),
            pl.BlockSpec((in_ch, h), lambda i: (0, 0)),
            pl.BlockSpec((NC, RB), lambda i: (0, i)),
        ],
        out_specs=pl.BlockSpec((RB, h), lambda i: (i, 0)),
        out_shape=jax.ShapeDtypeStruct((n_pad, h), jnp.float32),
    )


@functools.lru_cache(maxsize=None)
def _make_tc2(n, n_pad, h, o2):
    def body(p_ref, zp_ref, deg_ref, b1_ref, w_ref, tp_ref):
        dinv = _dinv_block(deg_ref[...], pl.program_id(0), n)
        hcur = dinv * (p_ref[0] + p_ref[1] + zp_ref[...]) + b1_ref[...]
        t = jnp.dot(hcur, w_ref[...], preferred_element_type=jnp.float32)
        tp_ref[...] = dinv * t

    return pl.pallas_call(
        body,
        grid=(n_pad // RB,),
        in_specs=[
            pl.BlockSpec((NC, RB, h), lambda i: (0, i, 0)),
            pl.BlockSpec((RB, h), lambda i: (i, 0)),
            pl.BlockSpec((NC, RB), lambda i: (0, i)),
            pl.BlockSpec((1, h), lambda i: (0, 0)),
            pl.BlockSpec((h, o2), lambda i: (0, 0)),
        ],
        out_specs=pl.BlockSpec((RB, o2), lambda i: (i, 0)),
        out_shape=jax.ShapeDtypeStruct((n_pad, o2), jnp.float32),
    )


@functools.lru_cache(maxsize=None)
def _make_tc3(n, n_pad, o):
    o2 = 2 * o

    def body(q_ref, tp_ref, deg_ref, b_ref, mu_ref, ls_ref):
        dinv = _dinv_block(deg_ref[...], pl.program_id(0), n)
        u = dinv * (q_ref[0] + q_ref[1] + tp_ref[...]) + b_ref[...]
        mu_ref[...] = u[:, :o]
        ls_ref[...] = u[:, o:]

    return pl.pallas_call(
        body,
        grid=(n_pad // RB,),
        in_specs=[
            pl.BlockSpec((NC, RB, o2), lambda i: (0, i, 0)),
            pl.BlockSpec((RB, o2), lambda i: (i, 0)),
            pl.BlockSpec((NC, RB), lambda i: (0, i)),
            pl.BlockSpec((1, o2), lambda i: (0, 0)),
        ],
        out_specs=[
            pl.BlockSpec((RB, o), lambda i: (i, 0)),
            pl.BlockSpec((RB, o), lambda i: (i, 0)),
        ],
        out_shape=[
            jax.ShapeDtypeStruct((n, o), jnp.float32),
            jax.ShapeDtypeStruct((n, o), jnp.float32),
        ],
    )


def kernel(x, edge_index, W1, b1, Wmu, bmu, Wls, bls):
    n, in_ch = x.shape
    h = W1.shape[1]
    o = Wmu.shape[1]
    o2 = 2 * o
    e = edge_index.shape[1]

    n_pad = -(-(n + 16) // (16 * NS)) * (16 * NS)
    n_win = -(-(-(-e // (NW * WIN))) // NBUF) * NBUF
    e_pad = n_win * NW * WIN

    padidx = n + (jnp.arange(e_pad - e, dtype=jnp.int32) % (n_pad - n))
    edges = jnp.concatenate(
        [edge_index, jnp.stack([padidx, padidx])], axis=1
    ).reshape(2, NW, n_win, WIN)
    x_p = jnp.pad(x, ((0, n_pad - n), (0, 0)))

    deg = _make_sc_degree(n_pad, n_win)(edges)

    zp = _make_tc1(n, n_pad, in_ch, h)(x_p, W1, deg)

    prop = _make_sc_propagate(n_pad, n_win, h)
    p = prop(zp, edges)

    w_cat = jnp.concatenate([Wmu, Wls], axis=1)
    tp = _make_tc2(n, n_pad, h, o2)(p, zp, deg, b1.reshape(1, h), w_cat)

    q = _make_sc_propagate(n_pad, n_win, o2)(tp, edges)

    b_cat = jnp.concatenate([bmu, bls]).reshape(1, o2)
    return tuple(_make_tc3(n, n_pad, o)(q, tp, deg, b_cat))
